# baseline (speedup 1.0000x reference)
import functools

import jax
import jax.numpy as jnp
from jax import lax
from jax.experimental import pallas as pl
from jax.experimental.pallas import tpu as pltpu
from jax.experimental.pallas import tpu_sc as plsc

N = 10000
E = 320000
IN = 128
HID = 256
OUT = 128

NC = 2
NS = 16
CHUNK = 64
NCH = 5120
E_PAD = NCH * CHUNK
PAD_ROWS = 64
N_ACC = N + PAD_ROWS
DEG_STRIPE = 632
N_DEG = NS * DEG_STRIPE

K1 = NCH // NS
K2 = NCH // (NC * NS)
IB1 = 64
IB2 = 32
NBUF = 4

_mesh = plsc.VectorSubcoreMesh(core_axis_name="c", subcore_axis_name="s")


@functools.partial(
    pl.kernel,
    out_type=(
        jax.ShapeDtypeStruct((N_DEG,), jnp.float32),
        jax.ShapeDtypeStruct((N_DEG,), jnp.float32),
    ),
    mesh=_mesh,
    scratch_types=[
        pltpu.VMEM_SHARED((N_DEG,), jnp.float32),
        pltpu.VMEM((K2, CHUNK), jnp.int32),
        pltpu.VMEM((CHUNK,), jnp.float32),
        pltpu.VMEM((DEG_STRIPE,), jnp.float32),
    ],
)
def _deg_kernel(dst_hbm, ones_hbm, zeros_hbm, out0_hbm, out1_hbm, acc, idx_v,
                ones_buf, stripe_buf):
    c = lax.axis_index("c")
    s = lax.axis_index("s")
    w = s * NC + c
    pltpu.sync_copy(dst_hbm.at[pl.ds(w * K2, K2)], idx_v)
    pltpu.sync_copy(ones_hbm, ones_buf)
    pltpu.sync_copy(zeros_hbm, stripe_buf)
    pltpu.sync_copy(stripe_buf, acc.at[pl.ds(s * DEG_STRIPE, DEG_STRIPE)])
    plsc.subcore_barrier()

    def step(k, carry):
        pltpu.sync_copy(ones_buf, acc.at[idx_v.at[k]], add=True)
        return carry

    lax.fori_loop(0, K2, step, 0)
    plsc.subcore_barrier()

    pltpu.sync_copy(acc.at[pl.ds(s * DEG_STRIPE, DEG_STRIPE)], stripe_buf)

    @pl.when(c == 0)
    def _():
        pltpu.sync_copy(stripe_buf, out0_hbm.at[pl.ds(s * DEG_STRIPE, DEG_STRIPE)])

    @pl.when(c == 1)
    def _():
        pltpu.sync_copy(stripe_buf, out1_hbm.at[pl.ds(s * DEG_STRIPE, DEG_STRIPE)])


_Z_STRIPE = 632
_Z_LAST = N_ACC - (NS - 1) * _Z_STRIPE
_W_STRIPE = 624
_W_LAST = N - (NS - 1) * _W_STRIPE


def _zero_acc(s, acc, zeros_hbm, stage):
    pltpu.sync_copy(zeros_hbm, stage)

    def zero_stripe(r0, nrows):
        nfull, rem = nrows // CHUNK, nrows % CHUNK
        for t in range(nfull):
            pltpu.sync_copy(stage, acc.at[pl.ds(r0 + t * CHUNK, CHUNK)])
        if rem:
            pltpu.sync_copy(stage.at[pl.ds(0, rem)],
                            acc.at[pl.ds(r0 + nfull * CHUNK, rem)])

    @pl.when(s < NS - 1)
    def _():
        zero_stripe(s * _Z_STRIPE, _Z_STRIPE)

    @pl.when(s == NS - 1)
    def _():
        zero_stripe((NS - 1) * _Z_STRIPE, _Z_LAST)


def _writeback(s, acc, out_hbm, stage):
    def wb(w0, nrows):
        nfull, rem = nrows // CHUNK, nrows % CHUNK
        for t in range(nfull):
            pltpu.sync_copy(acc.at[pl.ds(w0 + t * CHUNK, CHUNK)], stage)
            pltpu.sync_copy(stage, out_hbm.at[pl.ds(w0 + t * CHUNK, CHUNK)])
        if rem:
            pltpu.sync_copy(acc.at[pl.ds(w0 + nfull * CHUNK, rem)],
                            stage.at[pl.ds(0, rem)])
            pltpu.sync_copy(stage.at[pl.ds(0, rem)],
                            out_hbm.at[pl.ds(w0 + nfull * CHUNK, rem)])

    @pl.when(s < NS - 1)
    def _():
        wb(s * _W_STRIPE, _W_STRIPE)

    @pl.when(s == NS - 1)
    def _():
        wb((NS - 1) * _W_STRIPE, _W_LAST)


def _agg_loop(nb, ib, row0, src_hbm, dst_hbm, g_hbm, acc, src_v, dst_v,
              bufs, gsems, ssems):
    def gd(k, b):
        return pltpu.make_async_copy(g_hbm.at[src_v.at[k]], bufs[b], gsems[b])

    def sd_start(k, b):
        pltpu.async_copy(bufs[b], acc.at[dst_v.at[k]], ssems[b], add=True)

    def sd_wait(k, b):
        pltpu.make_async_copy(bufs[b], acc.at[dst_v.at[k]], ssems[b]).wait()

    def blk(j, carry):
        r = row0 + j * ib
        pltpu.sync_copy(src_hbm.at[pl.ds(r, ib)], src_v)
        pltpu.sync_copy(dst_hbm.at[pl.ds(r, ib)], dst_v)
        gd(0, 0).start()
        gd(1, 1).start()

        def body(k, b):
            gd(k, b).wait()
            sd_start(k, b)

            @pl.when(k + 2 < ib)
            def _():
                b2 = (b + 2) % NBUF

                @pl.when(k >= 2)
                def _():
                    sd_wait(k - 2, b2)

                gd(k + 2, b2).start()

        def step(k, c2):
            for b in range(NBUF):
                @pl.when(lax.rem(k, NBUF) == b)
                def _(b=b):
                    body(k, b)
            return c2

        lax.fori_loop(0, ib, step, 0)
        for t in range(NBUF):
            sd_wait(ib - NBUF + t, (ib - NBUF + t) % NBUF)
        return carry

    lax.fori_loop(0, nb, blk, 0)


@functools.partial(
    pl.kernel,
    out_type=(
        jax.ShapeDtypeStruct((N, HID // 2), jnp.float32),
        jax.ShapeDtypeStruct((N, HID // 2), jnp.float32),
    ),
    mesh=_mesh,
    scratch_types=[
        pltpu.VMEM_SHARED((N_ACC, HID // 2), jnp.float32),
        pltpu.VMEM((IB1, CHUNK), jnp.int32),
        pltpu.VMEM((IB1, CHUNK), jnp.int32),
    ] + [pltpu.VMEM((CHUNK, HID // 2), jnp.float32)] * NBUF
      + [pltpu.SemaphoreType.DMA] * (2 * NBUF),
)
def _agg_hid(g0_hbm, g1_hbm, src_hbm, dst_hbm, zeros_hbm, out0_hbm, out1_hbm,
             acc, src_v, dst_v, *bufs_sems):
    bufs = bufs_sems[:NBUF]
    gsems = bufs_sems[NBUF:2 * NBUF]
    ssems = bufs_sems[2 * NBUF:]
    c = lax.axis_index("c")
    s = lax.axis_index("s")
    _zero_acc(s, acc, zeros_hbm, bufs[0])
    plsc.subcore_barrier()

    def run(g_hbm, out_hbm):
        _agg_loop(K1 // IB1, IB1, s * K1, src_hbm, dst_hbm, g_hbm, acc,
                  src_v, dst_v, bufs, gsems, ssems)
        plsc.subcore_barrier()
        _writeback(s, acc, out_hbm, bufs[0])

    @pl.when(c == 0)
    def _():
        run(g0_hbm, out0_hbm)

    @pl.when(c == 1)
    def _():
        run(g1_hbm, out1_hbm)


@functools.partial(
    pl.kernel,
    out_type=(
        jax.ShapeDtypeStruct((N, OUT), jnp.float32),
        jax.ShapeDtypeStruct((N, OUT), jnp.float32),
    ),
    mesh=_mesh,
    scratch_types=[
        pltpu.VMEM_SHARED((N_ACC, OUT), jnp.float32),
        pltpu.VMEM((IB2, CHUNK), jnp.int32),
        pltpu.VMEM((IB2, CHUNK), jnp.int32),
    ] + [pltpu.VMEM((CHUNK, OUT), jnp.float32)] * NBUF
      + [pltpu.SemaphoreType.DMA] * (2 * NBUF),
)
def _agg_out(g_hbm, src_hbm, dst_hbm, zeros_hbm, out0_hbm, out1_hbm,
             acc, src_v, dst_v, *bufs_sems):
    bufs = bufs_sems[:NBUF]
    gsems = bufs_sems[NBUF:2 * NBUF]
    ssems = bufs_sems[2 * NBUF:]
    c = lax.axis_index("c")
    s = lax.axis_index("s")
    row0 = c * (NCH // NC) + s * K2
    _zero_acc(s, acc, zeros_hbm, bufs[0])
    plsc.subcore_barrier()

    _agg_loop(K2 // IB2, IB2, row0, src_hbm, dst_hbm, g_hbm, acc,
              src_v, dst_v, bufs, gsems, ssems)
    plsc.subcore_barrier()

    @pl.when(c == 0)
    def _():
        _writeback(s, acc, out0_hbm, bufs[0])

    @pl.when(c == 1)
    def _():
        _writeback(s, acc, out1_hbm, bufs[0])


_R = 2000


def _mm1_body(deg0_ref, deg1_ref, x_ref, w_ref, dinv_ref, g0_ref, g1_ref):
    dinv = lax.rsqrt(deg0_ref[...] + deg1_ref[...] + 1.0)
    h = jnp.dot(x_ref[...], w_ref[...], preferred_element_type=jnp.float32)
    g = h * dinv
    dinv_ref[...] = dinv
    g0_ref[...] = g[:, : HID // 2]
    g1_ref[...] = g[:, HID // 2:]


def _mm1_call(deg0, deg1, x, W1):
    grid = (N // _R,)
    return pl.pallas_call(
        _mm1_body,
        grid=grid,
        in_specs=[
            pl.BlockSpec((_R, 1), lambda i: (i, 0)),
            pl.BlockSpec((_R, 1), lambda i: (i, 0)),
            pl.BlockSpec((_R, IN), lambda i: (i, 0)),
            pl.BlockSpec((IN, HID), lambda i: (0, 0)),
        ],
        out_specs=[
            pl.BlockSpec((_R, 1), lambda i: (i, 0)),
            pl.BlockSpec((_R, HID // 2), lambda i: (i, 0)),
            pl.BlockSpec((_R, HID // 2), lambda i: (i, 0)),
        ],
        out_shape=[
            jax.ShapeDtypeStruct((N, 1), jnp.float32),
            jax.ShapeDtypeStruct((N, HID // 2), jnp.float32),
            jax.ShapeDtypeStruct((N, HID // 2), jnp.float32),
        ],
    )(deg0, deg1, x, W1)


def _mid_body(dinv_ref, s0_ref, s1_ref, g0_ref, g1_ref, b1_ref, w2_ref,
              emb_ref, g2_ref):
    dv = dinv_ref[...]
    b1 = b1_ref[...]
    e0 = (s0_ref[...] + g0_ref[...]) * dv + b1[None, : HID // 2]
    e1 = (s1_ref[...] + g1_ref[...]) * dv + b1[None, HID // 2:]
    emb = jnp.concatenate([e0, e1], axis=1)
    emb_ref[...] = emb
    h = jnp.maximum(emb, 0.0)
    mm2 = jnp.dot(h, w2_ref[...], preferred_element_type=jnp.float32)
    g2_ref[...] = mm2 * dv


def _mid_call(dinv, s0, s1, g0, g1, b1, W2):
    grid = (N // _R,)
    return pl.pallas_call(
        _mid_body,
        grid=grid,
        in_specs=[
            pl.BlockSpec((_R, 1), lambda i: (i, 0)),
            pl.BlockSpec((_R, HID // 2), lambda i: (i, 0)),
            pl.BlockSpec((_R, HID // 2), lambda i: (i, 0)),
            pl.BlockSpec((_R, HID // 2), lambda i: (i, 0)),
            pl.BlockSpec((_R, HID // 2), lambda i: (i, 0)),
            pl.BlockSpec((HID,), lambda i: (0,)),
            pl.BlockSpec((HID, OUT), lambda i: (0, 0)),
        ],
        out_specs=[
            pl.BlockSpec((_R, HID), lambda i: (i, 0)),
            pl.BlockSpec((_R, OUT), lambda i: (i, 0)),
        ],
        out_shape=[
            jax.ShapeDtypeStruct((N, HID), jnp.float32),
            jax.ShapeDtypeStruct((N, OUT), jnp.float32),
        ],
    )(dinv, s0, s1, g0, g1, b1, W2)


def _out_body(dinv_ref, s2a_ref, s2b_ref, g2_ref, b2_ref, out_ref):
    dv = dinv_ref[...]
    b2 = b2_ref[...]
    p = (s2a_ref[...] + s2b_ref[...] + g2_ref[...]) * dv + b2[None, :]
    m = jnp.max(p, axis=1, keepdims=True)
    lse = jnp.log(jnp.sum(jnp.exp(p - m), axis=1, keepdims=True)) + m
    out_ref[...] = p - lse


def _out_call(dinv, s2a, s2b, g2, b2):
    grid = (N // _R,)
    return pl.pallas_call(
        _out_body,
        grid=grid,
        in_specs=[
            pl.BlockSpec((_R, 1), lambda i: (i, 0)),
            pl.BlockSpec((_R, OUT), lambda i: (i, 0)),
            pl.BlockSpec((_R, OUT), lambda i: (i, 0)),
            pl.BlockSpec((_R, OUT), lambda i: (i, 0)),
            pl.BlockSpec((OUT,), lambda i: (0,)),
        ],
        out_specs=pl.BlockSpec((_R, OUT), lambda i: (i, 0)),
        out_shape=jax.ShapeDtypeStruct((N, OUT), jnp.float32),
    )(dinv, s2a, s2b, g2, b2)


def kernel(x, edge_index, W1, b1, W2, b2):
    src = edge_index[0]
    dst = edge_index[1]
    npad = E_PAD - E
    ar = jnp.arange(npad, dtype=jnp.int32)
    srcp = jnp.concatenate([src, ar % N]).reshape(NCH, CHUNK)
    dstp = jnp.concatenate([dst, N + (ar % PAD_ROWS)]).reshape(NCH, CHUNK)

    ones_c = jnp.ones((CHUNK,), jnp.float32)
    zeros_deg = jnp.zeros((DEG_STRIPE,), jnp.float32)
    zeros_h = jnp.zeros((CHUNK, HID // 2), jnp.float32)
    zeros_o = jnp.zeros((CHUNK, OUT), jnp.float32)

    dega, degb = _deg_kernel(dstp, ones_c, zeros_deg)
    dinv, g0, g1 = _mm1_call(dega[:N, None], degb[:N, None], x, W1)
    s0, s1 = _agg_hid(g0, g1, srcp, dstp, zeros_h)
    emb, g2 = _mid_call(dinv, s0, s1, g0, g1, b1, W2)
    s2a, s2b = _agg_out(g2, srcp, dstp, zeros_o)
    out = _out_call(dinv, s2a, s2b, g2, b2)
    return out, emb


# speedup vs baseline: 1.1022x; 1.1022x over previous
import functools

import jax
import jax.numpy as jnp
from jax import lax
from jax.experimental import pallas as pl
from jax.experimental.pallas import tpu as pltpu
from jax.experimental.pallas import tpu_sc as plsc

N = 10000
E = 320000
IN = 128
HID = 256
OUT = 128

NC = 2
NS = 16
CHUNK = 128
NCH = 2560
E_PAD = NCH * CHUNK
PAD_ROWS = 64
N_ACC = N + PAD_ROWS
DEG_STRIPE = 632
N_DEG = NS * DEG_STRIPE

K1 = NCH // NS
K2 = NCH // (NC * NS)
IB1 = 32
IB2 = 16
NBUF = 2

_mesh = plsc.VectorSubcoreMesh(core_axis_name="c", subcore_axis_name="s")


@functools.partial(
    pl.kernel,
    out_type=(
        jax.ShapeDtypeStruct((N_DEG,), jnp.float32),
        jax.ShapeDtypeStruct((N_DEG,), jnp.float32),
    ),
    mesh=_mesh,
    scratch_types=[
        pltpu.VMEM_SHARED((N_DEG,), jnp.float32),
        pltpu.VMEM((K2, CHUNK), jnp.int32),
        pltpu.VMEM((CHUNK,), jnp.float32),
        pltpu.VMEM((DEG_STRIPE,), jnp.float32),
    ],
)
def _deg_kernel(dst_hbm, ones_hbm, zeros_hbm, out0_hbm, out1_hbm, acc, idx_v,
                ones_buf, stripe_buf):
    c = lax.axis_index("c")
    s = lax.axis_index("s")
    w = s * NC + c
    pltpu.sync_copy(dst_hbm.at[pl.ds(w * K2, K2)], idx_v)
    pltpu.sync_copy(ones_hbm, ones_buf)
    pltpu.sync_copy(zeros_hbm, stripe_buf)
    pltpu.sync_copy(stripe_buf, acc.at[pl.ds(s * DEG_STRIPE, DEG_STRIPE)])
    plsc.subcore_barrier()

    def step(k, carry):
        pltpu.sync_copy(ones_buf, acc.at[idx_v.at[k]], add=True)
        return carry

    lax.fori_loop(0, K2, step, 0)
    plsc.subcore_barrier()

    pltpu.sync_copy(acc.at[pl.ds(s * DEG_STRIPE, DEG_STRIPE)], stripe_buf)

    @pl.when(c == 0)
    def _():
        pltpu.sync_copy(stripe_buf, out0_hbm.at[pl.ds(s * DEG_STRIPE, DEG_STRIPE)])

    @pl.when(c == 1)
    def _():
        pltpu.sync_copy(stripe_buf, out1_hbm.at[pl.ds(s * DEG_STRIPE, DEG_STRIPE)])


_Z_STRIPE = 632
_Z_LAST = N_ACC - (NS - 1) * _Z_STRIPE
_W_STRIPE = 624
_W_LAST = N - (NS - 1) * _W_STRIPE


def _zero_acc(s, acc, zeros_hbm, stage):
    pltpu.sync_copy(zeros_hbm, stage)

    def zero_stripe(r0, nrows):
        nfull, rem = nrows // CHUNK, nrows % CHUNK
        for t in range(nfull):
            pltpu.sync_copy(stage, acc.at[pl.ds(r0 + t * CHUNK, CHUNK)])
        if rem:
            pltpu.sync_copy(stage.at[pl.ds(0, rem)],
                            acc.at[pl.ds(r0 + nfull * CHUNK, rem)])

    @pl.when(s < NS - 1)
    def _():
        zero_stripe(s * _Z_STRIPE, _Z_STRIPE)

    @pl.when(s == NS - 1)
    def _():
        zero_stripe((NS - 1) * _Z_STRIPE, _Z_LAST)


def _writeback(s, acc, out_hbm, stage):
    def wb(w0, nrows):
        nfull, rem = nrows // CHUNK, nrows % CHUNK
        for t in range(nfull):
            pltpu.sync_copy(acc.at[pl.ds(w0 + t * CHUNK, CHUNK)], stage)
            pltpu.sync_copy(stage, out_hbm.at[pl.ds(w0 + t * CHUNK, CHUNK)])
        if rem:
            pltpu.sync_copy(acc.at[pl.ds(w0 + nfull * CHUNK, rem)],
                            stage.at[pl.ds(0, rem)])
            pltpu.sync_copy(stage.at[pl.ds(0, rem)],
                            out_hbm.at[pl.ds(w0 + nfull * CHUNK, rem)])

    @pl.when(s < NS - 1)
    def _():
        wb(s * _W_STRIPE, _W_STRIPE)

    @pl.when(s == NS - 1)
    def _():
        wb((NS - 1) * _W_STRIPE, _W_LAST)


def _agg_loop(nb, ib, row0, src_hbm, dst_hbm, g_hbm, acc, src_v, dst_v,
              bufs, gsems, ssems):
    del ssems

    def gd(k, b):
        return pltpu.make_async_copy(g_hbm.at[src_v.at[k]], bufs[b], gsems[b])

    def blk(j, carry):
        r = row0 + j * ib
        pltpu.sync_copy(src_hbm.at[pl.ds(r, ib)], src_v)
        pltpu.sync_copy(dst_hbm.at[pl.ds(r, ib)], dst_v)
        gd(0, 0).start()
        gd(1, 1).start()

        def body(k, b):
            gd(k, b).wait()
            pltpu.sync_copy(bufs[b], acc.at[dst_v.at[k]], add=True)

            @pl.when(k + 2 < ib)
            def _():
                gd(k + 2, b).start()

        def step(k, c2):
            @pl.when(lax.rem(k, 2) == 0)
            def _():
                body(k, 0)

            @pl.when(lax.rem(k, 2) == 1)
            def _():
                body(k, 1)

            return c2

        lax.fori_loop(0, ib, step, 0)
        return carry

    lax.fori_loop(0, nb, blk, 0)


@functools.partial(
    pl.kernel,
    out_type=(
        jax.ShapeDtypeStruct((N, HID // 2), jnp.float32),
        jax.ShapeDtypeStruct((N, HID // 2), jnp.float32),
    ),
    mesh=_mesh,
    scratch_types=[
        pltpu.VMEM_SHARED((N_ACC, HID // 2), jnp.float32),
        pltpu.VMEM((IB1, CHUNK), jnp.int32),
        pltpu.VMEM((IB1, CHUNK), jnp.int32),
    ] + [pltpu.VMEM((CHUNK, HID // 2), jnp.float32)] * NBUF
      + [pltpu.SemaphoreType.DMA] * (2 * NBUF),
)
def _agg_hid(g0_hbm, g1_hbm, src_hbm, dst_hbm, zeros_hbm, out0_hbm, out1_hbm,
             acc, src_v, dst_v, *bufs_sems):
    bufs = bufs_sems[:NBUF]
    gsems = bufs_sems[NBUF:2 * NBUF]
    ssems = bufs_sems[2 * NBUF:]
    c = lax.axis_index("c")
    s = lax.axis_index("s")
    _zero_acc(s, acc, zeros_hbm, bufs[0])
    plsc.subcore_barrier()

    def run(g_hbm, out_hbm):
        _agg_loop(K1 // IB1, IB1, s * K1, src_hbm, dst_hbm, g_hbm, acc,
                  src_v, dst_v, bufs, gsems, ssems)
        plsc.subcore_barrier()
        _writeback(s, acc, out_hbm, bufs[0])

    @pl.when(c == 0)
    def _():
        run(g0_hbm, out0_hbm)

    @pl.when(c == 1)
    def _():
        run(g1_hbm, out1_hbm)


@functools.partial(
    pl.kernel,
    out_type=(
        jax.ShapeDtypeStruct((N, OUT), jnp.float32),
        jax.ShapeDtypeStruct((N, OUT), jnp.float32),
    ),
    mesh=_mesh,
    scratch_types=[
        pltpu.VMEM_SHARED((N_ACC, OUT), jnp.float32),
        pltpu.VMEM((IB2, CHUNK), jnp.int32),
        pltpu.VMEM((IB2, CHUNK), jnp.int32),
    ] + [pltpu.VMEM((CHUNK, OUT), jnp.float32)] * NBUF
      + [pltpu.SemaphoreType.DMA] * (2 * NBUF),
)
def _agg_out(g_hbm, src_hbm, dst_hbm, zeros_hbm, out0_hbm, out1_hbm,
             acc, src_v, dst_v, *bufs_sems):
    bufs = bufs_sems[:NBUF]
    gsems = bufs_sems[NBUF:2 * NBUF]
    ssems = bufs_sems[2 * NBUF:]
    c = lax.axis_index("c")
    s = lax.axis_index("s")
    row0 = c * (NCH // NC) + s * K2
    _zero_acc(s, acc, zeros_hbm, bufs[0])
    plsc.subcore_barrier()

    _agg_loop(K2 // IB2, IB2, row0, src_hbm, dst_hbm, g_hbm, acc,
              src_v, dst_v, bufs, gsems, ssems)
    plsc.subcore_barrier()

    @pl.when(c == 0)
    def _():
        _writeback(s, acc, out0_hbm, bufs[0])

    @pl.when(c == 1)
    def _():
        _writeback(s, acc, out1_hbm, bufs[0])


_R = 2000


def _mm1_body(deg0_ref, deg1_ref, x_ref, w_ref, dinv_ref, g0_ref, g1_ref):
    dinv = lax.rsqrt(deg0_ref[...] + deg1_ref[...] + 1.0)
    h = jnp.dot(x_ref[...], w_ref[...], preferred_element_type=jnp.float32)
    g = h * dinv
    dinv_ref[...] = dinv
    g0_ref[...] = g[:, : HID // 2]
    g1_ref[...] = g[:, HID // 2:]


def _mm1_call(deg0, deg1, x, W1):
    grid = (N // _R,)
    return pl.pallas_call(
        _mm1_body,
        grid=grid,
        in_specs=[
            pl.BlockSpec((_R, 1), lambda i: (i, 0)),
            pl.BlockSpec((_R, 1), lambda i: (i, 0)),
            pl.BlockSpec((_R, IN), lambda i: (i, 0)),
            pl.BlockSpec((IN, HID), lambda i: (0, 0)),
        ],
        out_specs=[
            pl.BlockSpec((_R, 1), lambda i: (i, 0)),
            pl.BlockSpec((_R, HID // 2), lambda i: (i, 0)),
            pl.BlockSpec((_R, HID // 2), lambda i: (i, 0)),
        ],
        out_shape=[
            jax.ShapeDtypeStruct((N, 1), jnp.float32),
            jax.ShapeDtypeStruct((N, HID // 2), jnp.float32),
            jax.ShapeDtypeStruct((N, HID // 2), jnp.float32),
        ],
    )(deg0, deg1, x, W1)


def _mid_body(dinv_ref, s0_ref, s1_ref, g0_ref, g1_ref, b1_ref, w2_ref,
              emb_ref, g2_ref):
    dv = dinv_ref[...]
    b1 = b1_ref[...]
    e0 = (s0_ref[...] + g0_ref[...]) * dv + b1[None, : HID // 2]
    e1 = (s1_ref[...] + g1_ref[...]) * dv + b1[None, HID // 2:]
    emb = jnp.concatenate([e0, e1], axis=1)
    emb_ref[...] = emb
    h = jnp.maximum(emb, 0.0)
    mm2 = jnp.dot(h, w2_ref[...], preferred_element_type=jnp.float32)
    g2_ref[...] = mm2 * dv


def _mid_call(dinv, s0, s1, g0, g1, b1, W2):
    grid = (N // _R,)
    return pl.pallas_call(
        _mid_body,
        grid=grid,
        in_specs=[
            pl.BlockSpec((_R, 1), lambda i: (i, 0)),
            pl.BlockSpec((_R, HID // 2), lambda i: (i, 0)),
            pl.BlockSpec((_R, HID // 2), lambda i: (i, 0)),
            pl.BlockSpec((_R, HID // 2), lambda i: (i, 0)),
            pl.BlockSpec((_R, HID // 2), lambda i: (i, 0)),
            pl.BlockSpec((HID,), lambda i: (0,)),
            pl.BlockSpec((HID, OUT), lambda i: (0, 0)),
        ],
        out_specs=[
            pl.BlockSpec((_R, HID), lambda i: (i, 0)),
            pl.BlockSpec((_R, OUT), lambda i: (i, 0)),
        ],
        out_shape=[
            jax.ShapeDtypeStruct((N, HID), jnp.float32),
            jax.ShapeDtypeStruct((N, OUT), jnp.float32),
        ],
    )(dinv, s0, s1, g0, g1, b1, W2)


def _out_body(dinv_ref, s2a_ref, s2b_ref, g2_ref, b2_ref, out_ref):
    dv = dinv_ref[...]
    b2 = b2_ref[...]
    p = (s2a_ref[...] + s2b_ref[...] + g2_ref[...]) * dv + b2[None, :]
    m = jnp.max(p, axis=1, keepdims=True)
    lse = jnp.log(jnp.sum(jnp.exp(p - m), axis=1, keepdims=True)) + m
    out_ref[...] = p - lse


def _out_call(dinv, s2a, s2b, g2, b2):
    grid = (N // _R,)
    return pl.pallas_call(
        _out_body,
        grid=grid,
        in_specs=[
            pl.BlockSpec((_R, 1), lambda i: (i, 0)),
            pl.BlockSpec((_R, OUT), lambda i: (i, 0)),
            pl.BlockSpec((_R, OUT), lambda i: (i, 0)),
            pl.BlockSpec((_R, OUT), lambda i: (i, 0)),
            pl.BlockSpec((OUT,), lambda i: (0,)),
        ],
        out_specs=pl.BlockSpec((_R, OUT), lambda i: (i, 0)),
        out_shape=jax.ShapeDtypeStruct((N, OUT), jnp.float32),
    )(dinv, s2a, s2b, g2, b2)


def kernel(x, edge_index, W1, b1, W2, b2):
    src = edge_index[0]
    dst = edge_index[1]
    npad = E_PAD - E
    ar = jnp.arange(npad, dtype=jnp.int32)
    srcp = jnp.concatenate([src, ar % N]).reshape(NCH, CHUNK)
    dstp = jnp.concatenate([dst, N + (ar % PAD_ROWS)]).reshape(NCH, CHUNK)

    ones_c = jnp.ones((CHUNK,), jnp.float32)
    zeros_deg = jnp.zeros((DEG_STRIPE,), jnp.float32)
    zeros_h = jnp.zeros((CHUNK, HID // 2), jnp.float32)
    zeros_o = jnp.zeros((CHUNK, OUT), jnp.float32)

    dega, degb = _deg_kernel(dstp, ones_c, zeros_deg)
    dinv, g0, g1 = _mm1_call(dega[:, None], degb[:, None], x, W1)
    s0, s1 = _agg_hid(g0, g1, srcp, dstp, zeros_h)
    emb, g2 = _mid_call(dinv, s0, s1, g0, g1, b1, W2)
    s2a, s2b = _agg_out(g2, srcp, dstp, zeros_o)
    out = _out_call(dinv, s2a, s2b, g2, b2)
    return out, emb


# speedup vs baseline: 1.1690x; 1.0607x over previous
import functools

import jax
import jax.numpy as jnp
from jax import lax
from jax.experimental import pallas as pl
from jax.experimental.pallas import tpu as pltpu
from jax.experimental.pallas import tpu_sc as plsc

N = 10000
E = 320000
IN = 128
HID = 256
OUT = 128

NC = 2
NS = 16
CHUNK = 128
NCH = 2560
E_PAD = NCH * CHUNK
PAD_ROWS = 64
N_ACC = N + PAD_ROWS
DEG_STRIPE = 632
N_DEG = NS * DEG_STRIPE

K1 = NCH // NS
K2 = NCH // (NC * NS)
IB1 = 32
IB2 = 16
NBUF = 2

_mesh = plsc.VectorSubcoreMesh(core_axis_name="c", subcore_axis_name="s")


@functools.partial(
    pl.kernel,
    out_type=(
        jax.ShapeDtypeStruct((N_DEG,), jnp.float32),
        jax.ShapeDtypeStruct((N_DEG,), jnp.float32),
    ),
    mesh=_mesh,
    scratch_types=[
        pltpu.VMEM_SHARED((N_DEG,), jnp.float32),
        pltpu.VMEM((K2, CHUNK), jnp.int32),
        pltpu.VMEM((CHUNK,), jnp.float32),
        pltpu.VMEM((DEG_STRIPE,), jnp.float32),
    ],
)
def _deg_kernel(dst_hbm, ones_hbm, zeros_hbm, out0_hbm, out1_hbm, acc, idx_v,
                ones_buf, stripe_buf):
    c = lax.axis_index("c")
    s = lax.axis_index("s")
    w = s * NC + c
    pltpu.sync_copy(dst_hbm.at[pl.ds(w * K2, K2)], idx_v)
    pltpu.sync_copy(ones_hbm, ones_buf)
    pltpu.sync_copy(zeros_hbm, stripe_buf)
    pltpu.sync_copy(stripe_buf, acc.at[pl.ds(s * DEG_STRIPE, DEG_STRIPE)])
    plsc.subcore_barrier()

    def step(k, carry):
        pltpu.sync_copy(ones_buf, acc.at[idx_v.at[k]], add=True)
        return carry

    lax.fori_loop(0, K2, step, 0)
    plsc.subcore_barrier()

    pltpu.sync_copy(acc.at[pl.ds(s * DEG_STRIPE, DEG_STRIPE)], stripe_buf)

    @pl.when(c == 0)
    def _():
        pltpu.sync_copy(stripe_buf, out0_hbm.at[pl.ds(s * DEG_STRIPE, DEG_STRIPE)])

    @pl.when(c == 1)
    def _():
        pltpu.sync_copy(stripe_buf, out1_hbm.at[pl.ds(s * DEG_STRIPE, DEG_STRIPE)])


_Z_STRIPE = 632
_Z_LAST = N_ACC - (NS - 1) * _Z_STRIPE
_W_STRIPE = 624
_W_LAST = N - (NS - 1) * _W_STRIPE


def _zero_acc(s, acc, zeros_hbm, stage):
    pltpu.sync_copy(zeros_hbm, stage)

    def zero_stripe(r0, nrows):
        nfull, rem = nrows // CHUNK, nrows % CHUNK
        for t in range(nfull):
            pltpu.sync_copy(stage, acc.at[pl.ds(r0 + t * CHUNK, CHUNK)])
        if rem:
            pltpu.sync_copy(stage.at[pl.ds(0, rem)],
                            acc.at[pl.ds(r0 + nfull * CHUNK, rem)])

    @pl.when(s < NS - 1)
    def _():
        zero_stripe(s * _Z_STRIPE, _Z_STRIPE)

    @pl.when(s == NS - 1)
    def _():
        zero_stripe((NS - 1) * _Z_STRIPE, _Z_LAST)


def _writeback(s, acc, out_hbm, stage):
    def wb(w0, nrows):
        nfull, rem = nrows // CHUNK, nrows % CHUNK
        for t in range(nfull):
            pltpu.sync_copy(acc.at[pl.ds(w0 + t * CHUNK, CHUNK)], stage)
            pltpu.sync_copy(stage, out_hbm.at[pl.ds(w0 + t * CHUNK, CHUNK)])
        if rem:
            pltpu.sync_copy(acc.at[pl.ds(w0 + nfull * CHUNK, rem)],
                            stage.at[pl.ds(0, rem)])
            pltpu.sync_copy(stage.at[pl.ds(0, rem)],
                            out_hbm.at[pl.ds(w0 + nfull * CHUNK, rem)])

    @pl.when(s < NS - 1)
    def _():
        wb(s * _W_STRIPE, _W_STRIPE)

    @pl.when(s == NS - 1)
    def _():
        wb((NS - 1) * _W_STRIPE, _W_LAST)


def _agg_loop(kt, ib, row0, src_hbm, dst_hbm, g_hbm, acc, src_sets, dst_sets,
              bufs, gsems, isrcs, idsts):
    def gdesc(jp, q, bp):
        return pltpu.make_async_copy(g_hbm.at[src_sets[jp].at[q]], bufs[bp],
                                     gsems[bp])

    def idx_desc(j, jp):
        r = row0 + j * ib
        return (
            pltpu.make_async_copy(src_hbm.at[pl.ds(r, ib)], src_sets[jp],
                                  isrcs[jp]),
            pltpu.make_async_copy(dst_hbm.at[pl.ds(r, ib)], dst_sets[jp],
                                  idsts[jp]),
        )

    def body(k, bp, jp):
        q = lax.rem(k, ib)
        gdesc(jp, q, bp).wait()
        pltpu.sync_copy(bufs[bp], acc.at[dst_sets[jp].at[q]], add=True)

        @pl.when((q == 0) & (k + ib < kt))
        def _():
            for d in idx_desc(k // ib + 1, 1 - jp):
                d.start()

        @pl.when(k + 2 < kt)
        def _():
            q2 = lax.rem(k + 2, ib)

            @pl.when(q2 >= 2)
            def _():
                gdesc(jp, q2, bp).start()

            @pl.when(q2 == 0)
            def _():
                for d in idx_desc((k + 2) // ib, 1 - jp):
                    d.wait()
                gdesc(1 - jp, q2, bp).start()

            @pl.when(q2 == 1)
            def _():
                gdesc(1 - jp, q2, bp).start()

    def step(k, c2):
        for bp in range(2):
            for jp in range(2):
                @pl.when((lax.rem(k, 2) == bp) & (lax.rem(k // ib, 2) == jp))
                def _(bp=bp, jp=jp):
                    body(k, bp, jp)
        return c2

    for d in idx_desc(0, 0):
        d.start()
        d.wait()
    gdesc(0, 0, 0).start()
    gdesc(0, 1, 1).start()
    lax.fori_loop(0, kt, step, 0)


@functools.partial(
    pl.kernel,
    out_type=(
        jax.ShapeDtypeStruct((N, HID // 2), jnp.float32),
        jax.ShapeDtypeStruct((N, HID // 2), jnp.float32),
    ),
    mesh=_mesh,
    scratch_types=[
        pltpu.VMEM_SHARED((N_ACC, HID // 2), jnp.float32),
    ] + [pltpu.VMEM((IB1, CHUNK), jnp.int32)] * 4
      + [pltpu.VMEM((CHUNK, HID // 2), jnp.float32)] * 2
      + [pltpu.SemaphoreType.DMA] * 6,
)
def _agg_hid(g0_hbm, g1_hbm, src_hbm, dst_hbm, zeros_hbm, out0_hbm, out1_hbm,
             acc, sA, sB, dA, dB, buf0, buf1, g0s, g1s, ia, ib_, da, db):
    src_sets, dst_sets = (sA, sB), (dA, dB)
    bufs, gsems = (buf0, buf1), (g0s, g1s)
    isrcs, idsts = (ia, ib_), (da, db)
    c = lax.axis_index("c")
    s = lax.axis_index("s")
    _zero_acc(s, acc, zeros_hbm, bufs[0])
    plsc.subcore_barrier()

    def run(g_hbm, out_hbm):
        _agg_loop(K1, IB1, s * K1, src_hbm, dst_hbm, g_hbm, acc,
                  src_sets, dst_sets, bufs, gsems, isrcs, idsts)
        plsc.subcore_barrier()
        _writeback(s, acc, out_hbm, bufs[0])

    @pl.when(c == 0)
    def _():
        run(g0_hbm, out0_hbm)

    @pl.when(c == 1)
    def _():
        run(g1_hbm, out1_hbm)


@functools.partial(
    pl.kernel,
    out_type=(
        jax.ShapeDtypeStruct((N, OUT), jnp.float32),
        jax.ShapeDtypeStruct((N, OUT), jnp.float32),
    ),
    mesh=_mesh,
    scratch_types=[
        pltpu.VMEM_SHARED((N_ACC, OUT), jnp.float32),
    ] + [pltpu.VMEM((IB2, CHUNK), jnp.int32)] * 4
      + [pltpu.VMEM((CHUNK, OUT), jnp.float32)] * 2
      + [pltpu.SemaphoreType.DMA] * 6,
)
def _agg_out(g_hbm, src_hbm, dst_hbm, zeros_hbm, out0_hbm, out1_hbm,
             acc, sA, sB, dA, dB, buf0, buf1, g0s, g1s, ia, ib_, da, db):
    src_sets, dst_sets = (sA, sB), (dA, dB)
    bufs, gsems = (buf0, buf1), (g0s, g1s)
    isrcs, idsts = (ia, ib_), (da, db)
    c = lax.axis_index("c")
    s = lax.axis_index("s")
    row0 = c * (NCH // NC) + s * K2
    _zero_acc(s, acc, zeros_hbm, bufs[0])
    plsc.subcore_barrier()

    _agg_loop(K2, IB2, row0, src_hbm, dst_hbm, g_hbm, acc,
              src_sets, dst_sets, bufs, gsems, isrcs, idsts)
    plsc.subcore_barrier()

    @pl.when(c == 0)
    def _():
        _writeback(s, acc, out0_hbm, bufs[0])

    @pl.when(c == 1)
    def _():
        _writeback(s, acc, out1_hbm, bufs[0])


_R = 2000


def _mm1_body(deg0_ref, deg1_ref, x_ref, w_ref, dinv_ref, g0_ref, g1_ref):
    dinv = lax.rsqrt(deg0_ref[...] + deg1_ref[...] + 1.0)
    h = jnp.dot(x_ref[...], w_ref[...], preferred_element_type=jnp.float32)
    g = h * dinv
    dinv_ref[...] = dinv
    g0_ref[...] = g[:, : HID // 2]
    g1_ref[...] = g[:, HID // 2:]


def _mm1_call(deg0, deg1, x, W1):
    grid = (N // _R,)
    return pl.pallas_call(
        _mm1_body,
        grid=grid,
        in_specs=[
            pl.BlockSpec((_R, 1), lambda i: (i, 0)),
            pl.BlockSpec((_R, 1), lambda i: (i, 0)),
            pl.BlockSpec((_R, IN), lambda i: (i, 0)),
            pl.BlockSpec((IN, HID), lambda i: (0, 0)),
        ],
        out_specs=[
            pl.BlockSpec((_R, 1), lambda i: (i, 0)),
            pl.BlockSpec((_R, HID // 2), lambda i: (i, 0)),
            pl.BlockSpec((_R, HID // 2), lambda i: (i, 0)),
        ],
        out_shape=[
            jax.ShapeDtypeStruct((N, 1), jnp.float32),
            jax.ShapeDtypeStruct((N, HID // 2), jnp.float32),
            jax.ShapeDtypeStruct((N, HID // 2), jnp.float32),
        ],
    )(deg0, deg1, x, W1)


def _mid_body(dinv_ref, s0_ref, s1_ref, g0_ref, g1_ref, b1_ref, w2_ref,
              emb_ref, g2_ref):
    dv = dinv_ref[...]
    b1 = b1_ref[...]
    e0 = (s0_ref[...] + g0_ref[...]) * dv + b1[None, : HID // 2]
    e1 = (s1_ref[...] + g1_ref[...]) * dv + b1[None, HID // 2:]
    emb = jnp.concatenate([e0, e1], axis=1)
    emb_ref[...] = emb
    h = jnp.maximum(emb, 0.0)
    mm2 = jnp.dot(h, w2_ref[...], preferred_element_type=jnp.float32)
    g2_ref[...] = mm2 * dv


def _mid_call(dinv, s0, s1, g0, g1, b1, W2):
    grid = (N // _R,)
    return pl.pallas_call(
        _mid_body,
        grid=grid,
        in_specs=[
            pl.BlockSpec((_R, 1), lambda i: (i, 0)),
            pl.BlockSpec((_R, HID // 2), lambda i: (i, 0)),
            pl.BlockSpec((_R, HID // 2), lambda i: (i, 0)),
            pl.BlockSpec((_R, HID // 2), lambda i: (i, 0)),
            pl.BlockSpec((_R, HID // 2), lambda i: (i, 0)),
            pl.BlockSpec((HID,), lambda i: (0,)),
            pl.BlockSpec((HID, OUT), lambda i: (0, 0)),
        ],
        out_specs=[
            pl.BlockSpec((_R, HID), lambda i: (i, 0)),
            pl.BlockSpec((_R, OUT), lambda i: (i, 0)),
        ],
        out_shape=[
            jax.ShapeDtypeStruct((N, HID), jnp.float32),
            jax.ShapeDtypeStruct((N, OUT), jnp.float32),
        ],
    )(dinv, s0, s1, g0, g1, b1, W2)


def _out_body(dinv_ref, s2a_ref, s2b_ref, g2_ref, b2_ref, out_ref):
    dv = dinv_ref[...]
    b2 = b2_ref[...]
    p = (s2a_ref[...] + s2b_ref[...] + g2_ref[...]) * dv + b2[None, :]
    m = jnp.max(p, axis=1, keepdims=True)
    lse = jnp.log(jnp.sum(jnp.exp(p - m), axis=1, keepdims=True)) + m
    out_ref[...] = p - lse


def _out_call(dinv, s2a, s2b, g2, b2):
    grid = (N // _R,)
    return pl.pallas_call(
        _out_body,
        grid=grid,
        in_specs=[
            pl.BlockSpec((_R, 1), lambda i: (i, 0)),
            pl.BlockSpec((_R, OUT), lambda i: (i, 0)),
            pl.BlockSpec((_R, OUT), lambda i: (i, 0)),
            pl.BlockSpec((_R, OUT), lambda i: (i, 0)),
            pl.BlockSpec((OUT,), lambda i: (0,)),
        ],
        out_specs=pl.BlockSpec((_R, OUT), lambda i: (i, 0)),
        out_shape=jax.ShapeDtypeStruct((N, OUT), jnp.float32),
    )(dinv, s2a, s2b, g2, b2)


def kernel(x, edge_index, W1, b1, W2, b2):
    src = edge_index[0]
    dst = edge_index[1]
    npad = E_PAD - E
    ar = jnp.arange(npad, dtype=jnp.int32)
    srcp = jnp.concatenate([src, ar % N]).reshape(NCH, CHUNK)
    dstp = jnp.concatenate([dst, N + (ar % PAD_ROWS)]).reshape(NCH, CHUNK)

    ones_c = jnp.ones((CHUNK,), jnp.float32)
    zeros_deg = jnp.zeros((DEG_STRIPE,), jnp.float32)
    zeros_h = jnp.zeros((CHUNK, HID // 2), jnp.float32)
    zeros_o = jnp.zeros((CHUNK, OUT), jnp.float32)

    dega, degb = _deg_kernel(dstp, ones_c, zeros_deg)
    dinv, g0, g1 = _mm1_call(dega[:, None], degb[:, None], x, W1)
    s0, s1 = _agg_hid(g0, g1, srcp, dstp, zeros_h)
    emb, g2 = _mid_call(dinv, s0, s1, g0, g1, b1, W2)
    s2a, s2b = _agg_out(g2, srcp, dstp, zeros_o)
    out = _out_call(dinv, s2a, s2b, g2, b2)
    return out, emb
